# baseline (speedup 1.0000x reference)
import functools

import jax
import jax.numpy as jnp
from jax import lax
from jax.experimental import pallas as pl
from jax.experimental.pallas import tpu as pltpu
from jax.experimental.pallas import tpu_sc as plsc

N = 10000
E = 320000
C = 128

NPAD = 10016
CHUNK = 128
CPW = 160
INFLIGHT = 16


def _sc_masks(idx_all, zeros):
    mesh = plsc.VectorSubcoreMesh(core_axis_name="c", subcore_axis_name="s")

    @functools.partial(
        pl.kernel,
        out_type=jax.ShapeDtypeStruct((4, NPAD), jnp.float32),
        mesh=mesh,
        scratch_types=[
            pltpu.VMEM((CPW, CHUNK), jnp.int32),
            pltpu.VMEM((CHUNK,), jnp.float32),
            pltpu.VMEM_SHARED((NPAD,), jnp.float32),
            pltpu.VMEM_SHARED((NPAD,), jnp.float32),
            pltpu.SemaphoreType.DMA,
        ],
    )
    def k(idx_hbm, zeros_hbm, out_hbm, idx_v, ones_v, acc_a, acc_b, sem):
        c = lax.axis_index("c")
        s = lax.axis_index("s")

        @pl.when(s == 0)
        def _():
            pltpu.sync_copy(zeros_hbm, acc_a)

        @pl.when(s == 1)
        def _():
            pltpu.sync_copy(zeros_hbm, acc_b)

        for i in range(CHUNK // 16):
            ones_v[pl.ds(i * 16, 16)] = jnp.ones((16,), jnp.float32)

        plsc.subcore_barrier()

        for phase, acc in ((0, acc_a), (1, acc_b)):
            g = c * 2 + phase
            pltpu.sync_copy(idx_hbm.at[g, pl.ds(s * CPW, CPW)], idx_v)

            def body(i, carry, acc=acc):
                cps = [
                    pltpu.async_copy(ones_v, acc.at[idx_v.at[i * INFLIGHT + t]], sem)
                    for t in range(INFLIGHT)
                ]
                for cp in cps:
                    cp.wait()
                return carry

            lax.fori_loop(0, CPW // INFLIGHT, body, 0)

        plsc.subcore_barrier()

        @pl.when(s == 0)
        def _():
            pltpu.sync_copy(acc_a, out_hbm.at[c * 2])

        @pl.when(s == 1)
        def _():
            pltpu.sync_copy(acc_b, out_hbm.at[c * 2 + 1])

    return k(idx_all, zeros)


def _tc_body(f1, f2, m11, m12, m22, m21, wv1, bv1, wv2, bv2,
             wp1, bp1, wp2, bp2, o1, o2):
    v1 = jnp.dot(f1[...], wv1[...], preferred_element_type=jnp.float32) + bv1[...]
    v2 = jnp.dot(f2[...], wv2[...], preferred_element_type=jnp.float32) + bv2[...]
    w1 = wp1[...]
    w2 = wp2[...]
    o1[...] = (m11[...] * jnp.dot(v1, w1[:C], preferred_element_type=jnp.float32)
               + m12[...] * jnp.dot(v1, w1[C:], preferred_element_type=jnp.float32)
               + bp1[...])
    o2[...] = (m22[...] * jnp.dot(v2, w2[:C], preferred_element_type=jnp.float32)
               + m21[...] * jnp.dot(v2, w2[C:], preferred_element_type=jnp.float32)
               + bp2[...])


def kernel(feat1, coord1, graph1, feat2, coord2, graph2, graph12, graph21,
           Wq1, bq1, gq1, beq1, Wk1, bk1, gk1, bek1,
           Wq2, bq2, gq2, beq2, Wk2, bk2, gk2, bek2,
           Wv1, bv1, Wv2, bv2, Wp1, bp1, Wp2, bp2):
    def prep(g):
        x = g[0].astype(jnp.int32).reshape(E // CHUNK, CHUNK)
        pad = 16 * CPW - E // CHUNK
        return jnp.pad(x, ((0, pad), (0, 0)), constant_values=N)

    idx_all = jnp.stack([prep(graph1), prep(graph21), prep(graph2), prep(graph12)])
    zeros = jnp.zeros((NPAD,), jnp.float32)

    masks = _sc_masks(idx_all, zeros)
    mcol = masks[:, :N].reshape(4, N, 1)

    BR = 2000
    grid = (N // BR,)
    row = pl.BlockSpec((BR, C), lambda i: (i, 0))
    mask = pl.BlockSpec((BR, 1), lambda i: (i, 0))
    full = lambda *shape: pl.BlockSpec(shape, lambda i: tuple(0 for _ in shape))

    o1, o2 = pl.pallas_call(
        _tc_body,
        grid=grid,
        in_specs=[row, row, mask, mask, mask, mask,
                  full(C, C), full(1, C), full(C, C), full(1, C),
                  full(2 * C, C), full(1, C), full(2 * C, C), full(1, C)],
        out_specs=[row, row],
        out_shape=[jax.ShapeDtypeStruct((N, C), jnp.float32),
                   jax.ShapeDtypeStruct((N, C), jnp.float32)],
    )(feat1, feat2, mcol[0], mcol[1], mcol[2], mcol[3],
      Wv1, bv1.reshape(1, C), Wv2, bv2.reshape(1, C),
      Wp1, bp1.reshape(1, C), Wp2, bp2.reshape(1, C))
    return (o1, o2)


# speedup vs baseline: 1.3524x; 1.3524x over previous
import functools

import jax
import jax.numpy as jnp
from jax import lax
from jax.experimental import pallas as pl
from jax.experimental.pallas import tpu as pltpu
from jax.experimental.pallas import tpu_sc as plsc

N = 10000
E = 320000
C = 128

CHUNK = 64
ROWS = E // CHUNK
CPW = 320
STRIDE = 312
INFLIGHT = 16


def _sc_masks(g_a0, g_b0, g_a1, g_b1, zeros):
    mesh = plsc.VectorSubcoreMesh(core_axis_name="c", subcore_axis_name="s")

    @functools.partial(
        pl.kernel,
        out_type=[jax.ShapeDtypeStruct((N,), jnp.float32) for _ in range(4)],
        mesh=mesh,
        scratch_types=[
            pltpu.VMEM((CPW, CHUNK), jnp.int32),
            pltpu.VMEM((CHUNK,), jnp.float32),
            pltpu.VMEM_SHARED((N,), jnp.float32),
            pltpu.VMEM_SHARED((N,), jnp.float32),
            pltpu.SemaphoreType.DMA,
        ],
    )
    def k(ga0, gb0, ga1, gb1, zeros_hbm, out0, out1, out2, out3,
          idx_v, ones_v, acc_a, acc_b, sem):
        c = lax.axis_index("c")
        s = lax.axis_index("s")
        start = s * STRIDE

        def scatter_graph(g_hbm, acc):
            pltpu.sync_copy(g_hbm.at[0, pl.ds(start, CPW)], idx_v)

            def body(i, carry):
                cps = [
                    pltpu.async_copy(ones_v, acc.at[idx_v.at[i * INFLIGHT + t]], sem)
                    for t in range(INFLIGHT)
                ]
                for cp in cps:
                    cp.wait()
                return carry

            lax.fori_loop(0, CPW // INFLIGHT, body, 0)

        def run_core(ga, gb, outa, outb):
            @pl.when(s == 0)
            def _():
                pltpu.sync_copy(zeros_hbm, acc_a)

            @pl.when(s == 1)
            def _():
                pltpu.sync_copy(zeros_hbm, acc_b)

            plsc.subcore_barrier()
            scatter_graph(ga, acc_a)
            scatter_graph(gb, acc_b)
            plsc.subcore_barrier()

            @pl.when(s == 0)
            def _():
                pltpu.sync_copy(acc_a, outa)

            @pl.when(s == 1)
            def _():
                pltpu.sync_copy(acc_b, outb)

        for i in range(CHUNK // 16):
            ones_v[pl.ds(i * 16, 16)] = jnp.ones((16,), jnp.float32)

        @pl.when(c == 0)
        def _():
            run_core(ga0, gb0, out0, out1)

        @pl.when(c == 1)
        def _():
            run_core(ga1, gb1, out2, out3)

    return k(g_a0, g_b0, g_a1, g_b1, zeros)


def _tc_body(f1, f2, m11, m12, m22, m21, wv1, bv1, wv2, bv2,
             wp1, bp1, wp2, bp2, o1, o2):
    v1 = jnp.dot(f1[...], wv1[...], preferred_element_type=jnp.float32) + bv1[...]
    v2 = jnp.dot(f2[...], wv2[...], preferred_element_type=jnp.float32) + bv2[...]
    w1 = wp1[...]
    w2 = wp2[...]
    o1[...] = (m11[...] * jnp.dot(v1, w1[:C], preferred_element_type=jnp.float32)
               + m12[...] * jnp.dot(v1, w1[C:], preferred_element_type=jnp.float32)
               + bp1[...])
    o2[...] = (m22[...] * jnp.dot(v2, w2[:C], preferred_element_type=jnp.float32)
               + m21[...] * jnp.dot(v2, w2[C:], preferred_element_type=jnp.float32)
               + bp2[...])


def kernel(feat1, coord1, graph1, feat2, coord2, graph2, graph12, graph21,
           Wq1, bq1, gq1, beq1, Wk1, bk1, gk1, bek1,
           Wq2, bq2, gq2, beq2, Wk2, bk2, gk2, bek2,
           Wv1, bv1, Wv2, bv2, Wp1, bp1, Wp2, bp2):
    view = lambda g: g.astype(jnp.int32).reshape(2, ROWS, CHUNK)
    zeros = jnp.zeros((N,), jnp.float32)

    m11, m12, m22, m21 = _sc_masks(
        view(graph1), view(graph21), view(graph2), view(graph12), zeros)

    BR = 2000
    grid = (N // BR,)
    row = pl.BlockSpec((BR, C), lambda i: (i, 0))
    mask = pl.BlockSpec((BR, 1), lambda i: (i, 0))
    full = lambda *shape: pl.BlockSpec(shape, lambda i: tuple(0 for _ in shape))

    o1, o2 = pl.pallas_call(
        _tc_body,
        grid=grid,
        in_specs=[row, row, mask, mask, mask, mask,
                  full(C, C), full(1, C), full(C, C), full(1, C),
                  full(2 * C, C), full(1, C), full(2 * C, C), full(1, C)],
        out_specs=[row, row],
        out_shape=[jax.ShapeDtypeStruct((N, C), jnp.float32),
                   jax.ShapeDtypeStruct((N, C), jnp.float32)],
    )(feat1, feat2, m11.reshape(N, 1), m12.reshape(N, 1),
      m22.reshape(N, 1), m21.reshape(N, 1),
      Wv1, bv1.reshape(1, C), Wv2, bv2.reshape(1, C),
      Wp1, bp1.reshape(1, C), Wp2, bp2.reshape(1, C))
    return (o1, o2)


# speedup vs baseline: 1.4610x; 1.0803x over previous
import functools

import jax
import jax.numpy as jnp
from jax import lax
from jax.experimental import pallas as pl
from jax.experimental.pallas import tpu as pltpu
from jax.experimental.pallas import tpu_sc as plsc

N = 10000
E = 320000
C = 128

CHUNK = 64
ROWS = E // CHUNK
CPW = 320
STRIDE = 312
INFLIGHT = 16


def _sc_masks(g_a0, g_b0, g_a1, g_b1, zeros):
    mesh = plsc.VectorSubcoreMesh(core_axis_name="c", subcore_axis_name="s")

    @functools.partial(
        pl.kernel,
        out_type=[jax.ShapeDtypeStruct((N,), jnp.float32) for _ in range(4)],
        mesh=mesh,
        scratch_types=[
            pltpu.VMEM((CPW, CHUNK), jnp.int32),
            pltpu.VMEM((CHUNK,), jnp.float32),
            pltpu.VMEM_SHARED((N,), jnp.float32),
            pltpu.VMEM_SHARED((N,), jnp.float32),
            pltpu.SemaphoreType.DMA,
        ],
        compiler_params=pltpu.CompilerParams(use_tc_tiling_on_sc=False),
    )
    def k(ga0, gb0, ga1, gb1, zeros_hbm, out0, out1, out2, out3,
          idx_v, ones_v, acc_a, acc_b, sem):
        c = lax.axis_index("c")
        s = lax.axis_index("s")
        start = s * STRIDE

        def scatter_graph(g_hbm, acc):
            pltpu.sync_copy(g_hbm.at[0, pl.ds(start, CPW)], idx_v)

            def body(i, carry):
                cps = [
                    pltpu.async_copy(ones_v, acc.at[idx_v.at[i * INFLIGHT + t]], sem)
                    for t in range(INFLIGHT)
                ]
                for cp in cps:
                    cp.wait()
                return carry

            lax.fori_loop(0, CPW // INFLIGHT, body, 0)

        def run_core(ga, gb, outa, outb):
            @pl.when(s == 0)
            def _():
                pltpu.sync_copy(zeros_hbm, acc_a)

            @pl.when(s == 1)
            def _():
                pltpu.sync_copy(zeros_hbm, acc_b)

            plsc.subcore_barrier()
            scatter_graph(ga, acc_a)
            scatter_graph(gb, acc_b)
            plsc.subcore_barrier()

            @pl.when(s == 0)
            def _():
                pltpu.sync_copy(acc_a, outa)

            @pl.when(s == 1)
            def _():
                pltpu.sync_copy(acc_b, outb)

        for i in range(CHUNK // 16):
            ones_v[pl.ds(i * 16, 16)] = jnp.ones((16,), jnp.float32)

        @pl.when(c == 0)
        def _():
            run_core(ga0, gb0, out0, out1)

        @pl.when(c == 1)
        def _():
            run_core(ga1, gb1, out2, out3)

    return k(g_a0, g_b0, g_a1, g_b1, zeros)


def _tc_body(f1, f2, m11, m12, m22, m21, wv1, bv1, wv2, bv2,
             wp1, bp1, wp2, bp2, o1, o2):
    v1 = jnp.dot(f1[...], wv1[...], preferred_element_type=jnp.float32) + bv1[...]
    v2 = jnp.dot(f2[...], wv2[...], preferred_element_type=jnp.float32) + bv2[...]
    w1 = wp1[...]
    w2 = wp2[...]
    o1[...] = (m11[...] * jnp.dot(v1, w1[:C], preferred_element_type=jnp.float32)
               + m12[...] * jnp.dot(v1, w1[C:], preferred_element_type=jnp.float32)
               + bp1[...])
    o2[...] = (m22[...] * jnp.dot(v2, w2[:C], preferred_element_type=jnp.float32)
               + m21[...] * jnp.dot(v2, w2[C:], preferred_element_type=jnp.float32)
               + bp2[...])


def kernel(feat1, coord1, graph1, feat2, coord2, graph2, graph12, graph21,
           Wq1, bq1, gq1, beq1, Wk1, bk1, gk1, bek1,
           Wq2, bq2, gq2, beq2, Wk2, bk2, gk2, bek2,
           Wv1, bv1, Wv2, bv2, Wp1, bp1, Wp2, bp2):
    view = lambda g: g.astype(jnp.int32).reshape(2, ROWS, CHUNK)
    zeros = jnp.zeros((N,), jnp.float32)

    m11, m12, m22, m21 = _sc_masks(
        view(graph1), view(graph21), view(graph2), view(graph12), zeros)

    BR = 2000
    grid = (N // BR,)
    row = pl.BlockSpec((BR, C), lambda i: (i, 0))
    mask = pl.BlockSpec((BR, 1), lambda i: (i, 0))
    full = lambda *shape: pl.BlockSpec(shape, lambda i: tuple(0 for _ in shape))

    o1, o2 = pl.pallas_call(
        _tc_body,
        grid=grid,
        in_specs=[row, row, mask, mask, mask, mask,
                  full(C, C), full(1, C), full(C, C), full(1, C),
                  full(2 * C, C), full(1, C), full(2 * C, C), full(1, C)],
        out_specs=[row, row],
        out_shape=[jax.ShapeDtypeStruct((N, C), jnp.float32),
                   jax.ShapeDtypeStruct((N, C), jnp.float32)],
    )(feat1, feat2, m11.reshape(N, 1), m12.reshape(N, 1),
      m22.reshape(N, 1), m21.reshape(N, 1),
      Wv1, bv1.reshape(1, C), Wv2, bv2.reshape(1, C),
      Wp1, bp1.reshape(1, C), Wp2, bp2.reshape(1, C))
    return (o1, o2)


# speedup vs baseline: 1.9380x; 1.3265x over previous
import functools

import jax
import jax.numpy as jnp
from jax import lax
from jax.experimental import pallas as pl
from jax.experimental.pallas import tpu as pltpu
from jax.experimental.pallas import tpu_sc as plsc

N = 10000
E = 320000
C = 128

CHUNK = 64
ROWS = E // CHUNK
CPW = 320
STRIDE = 312
INFLIGHT = 16


def _sc_masks(g_a0, g_b0, g_a1, g_b1, zeros):
    mesh = plsc.VectorSubcoreMesh(core_axis_name="c", subcore_axis_name="s")

    @functools.partial(
        pl.kernel,
        out_type=jax.ShapeDtypeStruct((4, N), jnp.float32),
        mesh=mesh,
        scratch_types=[
            pltpu.VMEM((CPW, CHUNK), jnp.int32),
            pltpu.VMEM((CHUNK,), jnp.float32),
            pltpu.VMEM_SHARED((N,), jnp.float32),
            pltpu.VMEM_SHARED((N,), jnp.float32),
            pltpu.SemaphoreType.DMA,
        ],
        compiler_params=pltpu.CompilerParams(use_tc_tiling_on_sc=False),
    )
    def k(ga0, gb0, ga1, gb1, zeros_hbm, out, idx_v, ones_v, acc_a, acc_b, sem):
        c = lax.axis_index("c")
        s = lax.axis_index("s")
        start = s * STRIDE

        def scatter_graph(g_hbm, acc):
            pltpu.sync_copy(g_hbm.at[0, pl.ds(start, CPW)], idx_v)

            def body(i, carry):
                cps = [
                    pltpu.async_copy(ones_v, acc.at[idx_v.at[i * INFLIGHT + t]], sem)
                    for t in range(INFLIGHT)
                ]
                for cp in cps:
                    cp.wait()
                return carry

            lax.fori_loop(0, CPW // INFLIGHT, body, 0)

        def run_core(ga, gb, rowa, rowb):
            @pl.when(s == 0)
            def _():
                pltpu.sync_copy(zeros_hbm, acc_a)

            @pl.when(s == 1)
            def _():
                pltpu.sync_copy(zeros_hbm, acc_b)

            plsc.subcore_barrier()
            scatter_graph(ga, acc_a)
            scatter_graph(gb, acc_b)
            plsc.subcore_barrier()

            @pl.when(s == 0)
            def _():
                pltpu.sync_copy(acc_a, out.at[rowa])

            @pl.when(s == 1)
            def _():
                pltpu.sync_copy(acc_b, out.at[rowb])

        for i in range(CHUNK // 16):
            ones_v[pl.ds(i * 16, 16)] = jnp.ones((16,), jnp.float32)

        @pl.when(c == 0)
        def _():
            run_core(ga0, gb0, 0, 1)

        @pl.when(c == 1)
        def _():
            run_core(ga1, gb1, 2, 3)

    return k(g_a0, g_b0, g_a1, g_b1, zeros)


def _tc_body(f1, f2, masks, wv1, bv1, wv2, bv2,
             wp1, bp1, wp2, bp2, o1, o2):
    v1 = jnp.dot(f1[...], wv1[...], preferred_element_type=jnp.float32) + bv1[...]
    v2 = jnp.dot(f2[...], wv2[...], preferred_element_type=jnp.float32) + bv2[...]
    w1 = wp1[...]
    w2 = wp2[...]
    mt = jnp.transpose(masks[...])
    o1[...] = (mt[:, 0:1] * jnp.dot(v1, w1[:C], preferred_element_type=jnp.float32)
               + mt[:, 1:2] * jnp.dot(v1, w1[C:], preferred_element_type=jnp.float32)
               + bp1[...])
    o2[...] = (mt[:, 2:3] * jnp.dot(v2, w2[:C], preferred_element_type=jnp.float32)
               + mt[:, 3:4] * jnp.dot(v2, w2[C:], preferred_element_type=jnp.float32)
               + bp2[...])


def kernel(feat1, coord1, graph1, feat2, coord2, graph2, graph12, graph21,
           Wq1, bq1, gq1, beq1, Wk1, bk1, gk1, bek1,
           Wq2, bq2, gq2, beq2, Wk2, bk2, gk2, bek2,
           Wv1, bv1, Wv2, bv2, Wp1, bp1, Wp2, bp2):
    view = lambda g: g.astype(jnp.int32).reshape(2, ROWS, CHUNK)
    zeros = jnp.zeros((N,), jnp.float32)

    masks = _sc_masks(
        view(graph1), view(graph21), view(graph2), view(graph12), zeros)

    o1, o2 = pl.pallas_call(
        _tc_body,
        out_shape=[jax.ShapeDtypeStruct((N, C), jnp.float32),
                   jax.ShapeDtypeStruct((N, C), jnp.float32)],
    )(feat1, feat2, masks,
      Wv1, bv1.reshape(1, C), Wv2, bv2.reshape(1, C),
      Wp1, bp1.reshape(1, C), Wp2, bp2.reshape(1, C))
    return (o1, o2)


# speedup vs baseline: 2.5033x; 1.2917x over previous
import functools

import jax
import jax.numpy as jnp
from jax import lax
from jax.experimental import pallas as pl
from jax.experimental.pallas import tpu as pltpu
from jax.experimental.pallas import tpu_sc as plsc

N = 10000
E = 320000
C = 128

CHUNK = 128
EPW = 20480
ESTRIDE = 19968
INFLIGHT = 16


def _sc_masks(g_a0, g_b0, g_a1, g_b1, zeros):
    mesh = plsc.VectorSubcoreMesh(core_axis_name="c", subcore_axis_name="s")

    @functools.partial(
        pl.kernel,
        out_type=jax.ShapeDtypeStruct((4, N), jnp.float32),
        mesh=mesh,
        scratch_types=[
            pltpu.VMEM((EPW,), jnp.int32),
            pltpu.VMEM((CHUNK,), jnp.float32),
            pltpu.VMEM_SHARED((N,), jnp.float32),
            pltpu.VMEM_SHARED((N,), jnp.float32),
            pltpu.SemaphoreType.DMA,
        ],
    )
    def k(ga0, gb0, ga1, gb1, zeros_hbm, out, idx_v, ones_v, acc_a, acc_b, sem):
        c = lax.axis_index("c")
        s = lax.axis_index("s")
        start = s * ESTRIDE

        def scatter_graph(g_hbm, acc):
            pltpu.sync_copy(g_hbm.at[0, pl.ds(start, EPW)], idx_v)

            def body(i, carry):
                cps = [
                    pltpu.async_copy(
                        ones_v,
                        acc.at[idx_v.at[pl.ds((i * INFLIGHT + t) * CHUNK, CHUNK)]],
                        sem)
                    for t in range(INFLIGHT)
                ]
                for cp in cps:
                    cp.wait()
                return carry

            lax.fori_loop(0, EPW // CHUNK // INFLIGHT, body, 0)

        def run_core(ga, gb, rowa, rowb):
            @pl.when(s == 0)
            def _():
                pltpu.sync_copy(zeros_hbm, acc_a)

            @pl.when(s == 1)
            def _():
                pltpu.sync_copy(zeros_hbm, acc_b)

            plsc.subcore_barrier()
            scatter_graph(ga, acc_a)
            scatter_graph(gb, acc_b)
            plsc.subcore_barrier()

            @pl.when(s == 0)
            def _():
                pltpu.sync_copy(acc_a, out.at[rowa])

            @pl.when(s == 1)
            def _():
                pltpu.sync_copy(acc_b, out.at[rowb])

        for i in range(CHUNK // 16):
            ones_v[pl.ds(i * 16, 16)] = jnp.ones((16,), jnp.float32)

        @pl.when(c == 0)
        def _():
            run_core(ga0, gb0, 0, 1)

        @pl.when(c == 1)
        def _():
            run_core(ga1, gb1, 2, 3)

    return k(g_a0, g_b0, g_a1, g_b1, zeros)


def _tc_body(f1, f2, masks, wv1, bv1, wv2, bv2,
             wp1, bp1, wp2, bp2, o1, o2):
    v1 = jnp.dot(f1[...], wv1[...], preferred_element_type=jnp.float32) + bv1[...]
    v2 = jnp.dot(f2[...], wv2[...], preferred_element_type=jnp.float32) + bv2[...]
    w1 = wp1[...]
    w2 = wp2[...]
    mt = jnp.transpose(masks[...])
    o1[...] = (mt[:, 0:1] * jnp.dot(v1, w1[:C], preferred_element_type=jnp.float32)
               + mt[:, 1:2] * jnp.dot(v1, w1[C:], preferred_element_type=jnp.float32)
               + bp1[...])
    o2[...] = (mt[:, 2:3] * jnp.dot(v2, w2[:C], preferred_element_type=jnp.float32)
               + mt[:, 3:4] * jnp.dot(v2, w2[C:], preferred_element_type=jnp.float32)
               + bp2[...])


def kernel(feat1, coord1, graph1, feat2, coord2, graph2, graph12, graph21,
           Wq1, bq1, gq1, beq1, Wk1, bk1, gk1, bek1,
           Wq2, bq2, gq2, beq2, Wk2, bk2, gk2, bek2,
           Wv1, bv1, Wv2, bv2, Wp1, bp1, Wp2, bp2):
    view = lambda g: g.astype(jnp.int32)
    zeros = jnp.zeros((N,), jnp.float32)

    masks = _sc_masks(
        view(graph1), view(graph21), view(graph2), view(graph12), zeros)

    o1, o2 = pl.pallas_call(
        _tc_body,
        out_shape=[jax.ShapeDtypeStruct((N, C), jnp.float32),
                   jax.ShapeDtypeStruct((N, C), jnp.float32)],
    )(feat1, feat2, masks,
      Wv1, bv1.reshape(1, C), Wv2, bv2.reshape(1, C),
      Wp1, bp1.reshape(1, C), Wp2, bp2.reshape(1, C))
    return (o1, o2)
